# baseline (speedup 1.0000x reference)
import jax
import jax.numpy as jnp
from jax.experimental import pallas as pl
from jax.experimental.pallas import tpu as pltpu

_N_CHUNKS = 16


def _body(w_hbm, o_hbm, vbuf, in_sems, out_sem):
    seq = vbuf.shape[0]
    batch = o_hbm.shape[0]
    blk = seq // _N_CHUNKS
    cins = []
    for c in range(_N_CHUNKS):
        rows = pl.ds(c * blk, blk)
        cp = pltpu.make_async_copy(w_hbm.at[rows, :], vbuf.at[rows, :],
                                   in_sems.at[c])
        cp.start()
        cins.append(cp)
    couts = []
    for c in range(_N_CHUNKS):
        cins[c].wait()
        rows = pl.ds(c * blk, blk)
        for b in range(batch):
            cp = pltpu.make_async_copy(vbuf.at[rows, :],
                                       o_hbm.at[b, rows, :], out_sem)
            cp.start()
            couts.append(cp)
    for cp in couts:
        cp.wait()


def kernel(tokens, W_pos):
    batch, seq = tokens.shape
    d_model = W_pos.shape[-1]
    return pl.pallas_call(
        _body,
        in_specs=[pl.BlockSpec(memory_space=pl.ANY)],
        out_specs=pl.BlockSpec(memory_space=pl.ANY),
        out_shape=jax.ShapeDtypeStruct((batch, seq, d_model), W_pos.dtype),
        scratch_shapes=[
            pltpu.VMEM((seq, d_model), W_pos.dtype),
            pltpu.SemaphoreType.DMA((_N_CHUNKS,)),
            pltpu.SemaphoreType.DMA,
        ],
    )(W_pos)


# speedup vs baseline: 1.0247x; 1.0247x over previous
import jax
import jax.numpy as jnp
from jax.experimental import pallas as pl
from jax.experimental.pallas import tpu as pltpu

_N_CHUNKS = 4


def _body(w_hbm, o_hbm, vbuf, in_sems, out_sem):
    seq = vbuf.shape[0]
    batch = o_hbm.shape[0]
    blk = seq // _N_CHUNKS
    cins = []
    for c in range(_N_CHUNKS):
        rows = pl.ds(c * blk, blk)
        cp = pltpu.make_async_copy(w_hbm.at[rows, :], vbuf.at[rows, :],
                                   in_sems.at[c])
        cp.start()
        cins.append(cp)
    couts = []
    for c in range(_N_CHUNKS):
        cins[c].wait()
        rows = pl.ds(c * blk, blk)
        for b in range(batch):
            cp = pltpu.make_async_copy(vbuf.at[rows, :],
                                       o_hbm.at[b, rows, :], out_sem)
            cp.start()
            couts.append(cp)
    for cp in couts:
        cp.wait()


def kernel(tokens, W_pos):
    batch, seq = tokens.shape
    d_model = W_pos.shape[-1]
    return pl.pallas_call(
        _body,
        in_specs=[pl.BlockSpec(memory_space=pl.ANY)],
        out_specs=pl.BlockSpec(memory_space=pl.ANY),
        out_shape=jax.ShapeDtypeStruct((batch, seq, d_model), W_pos.dtype),
        scratch_shapes=[
            pltpu.VMEM((seq, d_model), W_pos.dtype),
            pltpu.SemaphoreType.DMA((_N_CHUNKS,)),
            pltpu.SemaphoreType.DMA,
        ],
    )(W_pos)
